# baseline (speedup 1.0000x reference)
import functools

import jax
import jax.numpy as jnp
from jax import lax
from jax.experimental import pallas as pl
from jax.experimental.pallas import tpu as pltpu
from jax.experimental.pallas import tpu_sc as plsc

BI = 128


def _select_body(r_ref, colx, coly, colz, rowx, rowy, rowz, xyzmat_ref,
                 idx_ref, gidx_ref, gxyz_ref):
    b = pl.program_id(0)
    ib = pl.program_id(1)
    r = r_ref[0, 0]
    n = rowx.shape[-1]
    bi = colx.shape[-2]

    xi = colx[0]
    yi = coly[0]
    zi = colz[0]
    xj = rowx[0]
    yj = rowy[0]
    zj = rowz[0]

    dx = xj - xi
    dy = yj - yi
    dz = zj - zi
    dist2 = dx * dx + dy * dy + dz * dz

    jota = lax.broadcasted_iota(jnp.int32, (bi, n), 1)
    iglob = lax.broadcasted_iota(jnp.int32, (bi, n), 0) + ib * bi
    within = ((jnp.abs(dx) < r) & (jnp.abs(dy) < r) & (jnp.abs(dz) < r)
              & (jota != iglob))
    inf = jnp.float32(1e10)
    d_in = jnp.where(within, dist2, inf)
    octant = ((dx > 0).astype(jnp.int32) * 4
              + (dy > 0).astype(jnp.int32) * 2
              + (dz > 0).astype(jnp.int32))
    iglob_col = lax.broadcasted_iota(jnp.int32, (bi, 1), 0) + ib * bi
    xyzmat = xyzmat_ref[0]
    center = jnp.concatenate([xi, yi, zi], axis=1)

    for o in range(8):
        d_o = jnp.where(octant == o, d_in, inf)
        mn = jnp.min(d_o, axis=1, keepdims=True)
        cand = jnp.min(jnp.where(d_o == mn, jota, n), axis=1, keepdims=True)
        sel = jnp.where(mn < 1e9, cand, iglob_col)
        idx_ref[0, :, o:o + 1] = sel
        gidx_ref[0, :, o:o + 1] = sel + b * n
        onehot = (jota == sel).astype(jnp.float32)
        g = jax.lax.dot_general(
            onehot, xyzmat, (((1,), (0,)), ((), ())),
            precision=jax.lax.Precision.HIGHEST,
            preferred_element_type=jnp.float32)
        gxyz_ref[0, :, 3 * o:3 * o + 3] = g - center


def _select_cube_tc(xyz, radius):
    B, N, _ = xyz.shape
    x = xyz[:, :, 0]
    y = xyz[:, :, 1]
    z = xyz[:, :, 2]
    col = lambda a: a[:, :, None]
    row = lambda a: a[:, None, :]
    r2 = jnp.reshape(radius, (1, 1))

    col_spec = pl.BlockSpec((1, BI, 1), lambda b, i: (b, i, 0))
    row_spec = pl.BlockSpec((1, 1, N), lambda b, i: (b, 0, 0))
    grid = (B, N // BI)
    idx, gidx, gxyz24 = pl.pallas_call(
        _select_body,
        grid=grid,
        in_specs=[
            pl.BlockSpec(memory_space=pltpu.SMEM),
            col_spec, col_spec, col_spec,
            row_spec, row_spec, row_spec,
            pl.BlockSpec((1, N, 3), lambda b, i: (b, 0, 0)),
        ],
        out_specs=[
            pl.BlockSpec((1, BI, 8), lambda b, i: (b, i, 0)),
            pl.BlockSpec((1, BI, 8), lambda b, i: (b, i, 0)),
            pl.BlockSpec((1, BI, 24), lambda b, i: (b, i, 0)),
        ],
        out_shape=[
            jax.ShapeDtypeStruct((B, N, 8), jnp.int32),
            jax.ShapeDtypeStruct((B, N, 8), jnp.int32),
            jax.ShapeDtypeStruct((B, N, 24), jnp.float32),
        ],
    )(r2, col(x), col(y), col(z), row(x), row(y), row(z), xyz)
    return idx, gidx, gxyz24


def _sc_gather(table, idxs):
    R, D = table.shape
    M = idxs.shape[0]
    info = plsc.get_sparse_core_info()
    nw = info.num_cores * info.num_subcores
    per_w = M // nw
    CH = 128
    T = per_w // CH
    idx3 = idxs.reshape(nw, T, CH)
    mesh = plsc.VectorSubcoreMesh(core_axis_name="c", subcore_axis_name="s")

    @functools.partial(
        pl.kernel,
        mesh=mesh,
        compiler_params=pltpu.CompilerParams(use_tc_tiling_on_sc=False),
        out_type=jax.ShapeDtypeStruct((M, D), jnp.float32),
        scratch_types=[
            pltpu.VMEM((T, CH), jnp.int32),
            pltpu.VMEM((CH, D), jnp.float32),
            pltpu.VMEM((CH, D), jnp.float32),
            pltpu.SemaphoreType.DMA,
            pltpu.SemaphoreType.DMA,
        ],
    )
    def k(idx_hbm, table_hbm, out_hbm, idx_v, rows0, rows1, sem0, sem1):
        wid = lax.axis_index("s") * info.num_cores + lax.axis_index("c")
        pltpu.sync_copy(idx_hbm.at[wid], idx_v)
        bufs = (rows0, rows1)
        sems = (sem0, sem1)
        cps = [None, None]
        cps[0] = pltpu.async_copy(table_hbm.at[idx_v.at[0]], bufs[0], sems[0])
        for t in range(T):
            cur = t % 2
            nxt = (t + 1) % 2
            if t + 1 < T:
                cps[nxt] = pltpu.async_copy(
                    table_hbm.at[idx_v.at[t + 1]], bufs[nxt], sems[nxt])
            cps[cur].wait()
            pltpu.sync_copy(bufs[cur], out_hbm.at[pl.ds(wid * per_w + t * CH, CH)])

    return k(idx3, table)


def kernel(xyz, points, radius):
    B, N, _ = xyz.shape
    P = points.shape[-1]
    idx, gidx, gxyz24 = _select_cube_tc(xyz, radius)
    grouped_xyz = gxyz24.reshape(B, N, 8, 3)
    gp = _sc_gather(points.reshape(B * N, P), gidx.reshape(B * N * 8))
    grouped_points = jnp.concatenate(
        [grouped_xyz, gp.reshape(B, N, 8, P)], axis=-1)
    return grouped_xyz, grouped_points, idx


# speedup vs baseline: 1.4831x; 1.4831x over previous
import functools

import jax
import jax.numpy as jnp
from jax import lax
from jax.experimental import pallas as pl
from jax.experimental.pallas import tpu as pltpu
from jax.experimental.pallas import tpu_sc as plsc

BI = 128


def _select_body(r_ref, colx, coly, colz, rowx, rowy, rowz,
                 idx_ref, gidx_ref, gxyz_ref):
    b = pl.program_id(0)
    ib = pl.program_id(1)
    r = r_ref[0, 0]
    n = rowx.shape[-1]
    bi = colx.shape[-2]

    xi = colx[0]
    yi = coly[0]
    zi = colz[0]
    xj = rowx[0]
    yj = rowy[0]
    zj = rowz[0]

    dx = xj - xi
    dy = yj - yi
    dz = zj - zi
    dist2 = dx * dx + dy * dy + dz * dz

    jota = lax.broadcasted_iota(jnp.int32, (bi, n), 1)
    iglob = lax.broadcasted_iota(jnp.int32, (bi, n), 0) + ib * bi
    within = ((jnp.abs(dx) < r) & (jnp.abs(dy) < r) & (jnp.abs(dz) < r)
              & (jota != iglob))
    inf = jnp.float32(1e10)
    d_in = jnp.where(within, dist2, inf)
    iglob_col = lax.broadcasted_iota(jnp.int32, (bi, 1), 0) + ib * bi

    sx = dx > 0
    sy = dy > 0
    sz = dz > 0
    dz1 = jnp.where(sz, d_in, inf)
    dz0 = jnp.where(sz, inf, d_in)
    dzy = [jnp.where(sy, inf, dz0), jnp.where(sy, inf, dz1),
           jnp.where(sy, dz0, inf), jnp.where(sy, dz1, inf)]

    for o in range(8):
        ox, orest = o // 4, o % 4
        base_d = dzy[orest]
        d_o = jnp.where(sx, base_d, inf) if ox else jnp.where(sx, inf, base_d)
        mn = jnp.min(d_o, axis=1, keepdims=True)
        cand = jnp.min(jnp.where(d_o == mn, jota, n), axis=1, keepdims=True)
        sel = jnp.where(mn < 1e9, cand, iglob_col)
        idx_ref[0, :, o:o + 1] = sel
        gidx_ref[0, :, o:o + 1] = sel + b * n
        p = jota == sel
        gxyz_ref[0, :, 3 * o:3 * o + 1] = jnp.sum(
            jnp.where(p, dx, 0.0), axis=1, keepdims=True)
        gxyz_ref[0, :, 3 * o + 1:3 * o + 2] = jnp.sum(
            jnp.where(p, dy, 0.0), axis=1, keepdims=True)
        gxyz_ref[0, :, 3 * o + 2:3 * o + 3] = jnp.sum(
            jnp.where(p, dz, 0.0), axis=1, keepdims=True)


def _select_cube_tc(xyz, radius):
    B, N, _ = xyz.shape
    x = xyz[:, :, 0]
    y = xyz[:, :, 1]
    z = xyz[:, :, 2]
    col = lambda a: a[:, :, None]
    row = lambda a: a[:, None, :]
    r2 = jnp.reshape(radius, (1, 1))

    col_spec = pl.BlockSpec((1, BI, 1), lambda b, i: (b, i, 0))
    row_spec = pl.BlockSpec((1, 1, N), lambda b, i: (b, 0, 0))
    grid = (B, N // BI)
    idx, gidx, gxyz24 = pl.pallas_call(
        _select_body,
        grid=grid,
        in_specs=[
            pl.BlockSpec(memory_space=pltpu.SMEM),
            col_spec, col_spec, col_spec,
            row_spec, row_spec, row_spec,
        ],
        out_specs=[
            pl.BlockSpec((1, BI, 8), lambda b, i: (b, i, 0)),
            pl.BlockSpec((1, BI, 8), lambda b, i: (b, i, 0)),
            pl.BlockSpec((1, BI, 24), lambda b, i: (b, i, 0)),
        ],
        out_shape=[
            jax.ShapeDtypeStruct((B, N, 8), jnp.int32),
            jax.ShapeDtypeStruct((B, N, 8), jnp.int32),
            jax.ShapeDtypeStruct((B, N, 24), jnp.float32),
        ],
    )(r2, col(x), col(y), col(z), row(x), row(y), row(z))
    return idx, gidx, gxyz24


def _sc_gather(table, idxs):
    R, D = table.shape
    M = idxs.shape[0]
    info = plsc.get_sparse_core_info()
    nw = info.num_cores * info.num_subcores
    per_w = M // nw
    CH = 128
    T = per_w // CH
    idx3 = idxs.reshape(nw, T, CH)
    mesh = plsc.VectorSubcoreMesh(core_axis_name="c", subcore_axis_name="s")

    @functools.partial(
        pl.kernel,
        mesh=mesh,
        compiler_params=pltpu.CompilerParams(use_tc_tiling_on_sc=False),
        out_type=jax.ShapeDtypeStruct((M, D), jnp.float32),
        scratch_types=[
            pltpu.VMEM((T, CH), jnp.int32),
            pltpu.VMEM((CH, D), jnp.float32),
            pltpu.VMEM((CH, D), jnp.float32),
            pltpu.SemaphoreType.DMA,
            pltpu.SemaphoreType.DMA,
        ],
    )
    def k(idx_hbm, table_hbm, out_hbm, idx_v, rows0, rows1, sem0, sem1):
        wid = lax.axis_index("s") * info.num_cores + lax.axis_index("c")
        pltpu.sync_copy(idx_hbm.at[wid], idx_v)
        bufs = (rows0, rows1)
        sems = (sem0, sem1)
        cps = [None, None]
        cps[0] = pltpu.async_copy(table_hbm.at[idx_v.at[0]], bufs[0], sems[0])
        for t in range(T):
            cur = t % 2
            nxt = (t + 1) % 2
            if t + 1 < T:
                cps[nxt] = pltpu.async_copy(
                    table_hbm.at[idx_v.at[t + 1]], bufs[nxt], sems[nxt])
            cps[cur].wait()
            pltpu.sync_copy(bufs[cur], out_hbm.at[pl.ds(wid * per_w + t * CH, CH)])

    return k(idx3, table)


def kernel(xyz, points, radius):
    B, N, _ = xyz.shape
    P = points.shape[-1]
    idx, gidx, gxyz24 = _select_cube_tc(xyz, radius)
    grouped_xyz = gxyz24.reshape(B, N, 8, 3)
    gp = _sc_gather(points.reshape(B * N, P), gidx.reshape(B * N * 8))
    grouped_points = jnp.concatenate(
        [grouped_xyz, gp.reshape(B, N, 8, P)], axis=-1)
    return grouped_xyz, grouped_points, idx


# speedup vs baseline: 1.5751x; 1.0620x over previous
import functools

import jax
import jax.numpy as jnp
from jax import lax
from jax.experimental import pallas as pl
from jax.experimental.pallas import tpu as pltpu
from jax.experimental.pallas import tpu_sc as plsc

BI = 256


def _select_body(r_ref, colx, coly, colz, rowx, rowy, rowz,
                 idx_ref, gidx_ref, gxyz_ref):
    b = pl.program_id(0)
    ib = pl.program_id(1)
    r = r_ref[0, 0]
    n = rowx.shape[-1]
    bi = colx.shape[-2]

    xi = colx[0]
    yi = coly[0]
    zi = colz[0]
    xj = rowx[0]
    yj = rowy[0]
    zj = rowz[0]

    dx = xj - xi
    dy = yj - yi
    dz = zj - zi
    dist2 = dx * dx + dy * dy + dz * dz

    jota = lax.broadcasted_iota(jnp.int32, (bi, n), 1)
    iglob = lax.broadcasted_iota(jnp.int32, (bi, n), 0) + ib * bi
    within = ((jnp.abs(dx) < r) & (jnp.abs(dy) < r) & (jnp.abs(dz) < r)
              & (jota != iglob))
    inf = jnp.float32(1e10)
    d_in = jnp.where(within, dist2, inf)
    iglob_col = lax.broadcasted_iota(jnp.int32, (bi, 1), 0) + ib * bi
    octant = ((dx > 0).astype(jnp.int32) * 4
              + (dy > 0).astype(jnp.int32) * 2
              + (dz > 0).astype(jnp.int32))

    for o in range(8):
        d_o = jnp.where(octant == o, d_in, inf)
        mn = jnp.min(d_o, axis=1, keepdims=True)
        cand = jnp.min(jnp.where(d_o == mn, jota, n), axis=1, keepdims=True)
        sel = jnp.where(mn < 1e9, cand, iglob_col)
        idx_ref[0, :, o:o + 1] = sel
        gidx_ref[0, :, o:o + 1] = sel + b * n
        p = jota == sel
        gxyz_ref[0, :, 3 * o:3 * o + 1] = jnp.sum(
            jnp.where(p, dx, 0.0), axis=1, keepdims=True)
        gxyz_ref[0, :, 3 * o + 1:3 * o + 2] = jnp.sum(
            jnp.where(p, dy, 0.0), axis=1, keepdims=True)
        gxyz_ref[0, :, 3 * o + 2:3 * o + 3] = jnp.sum(
            jnp.where(p, dz, 0.0), axis=1, keepdims=True)


def _select_cube_tc(xyz, radius):
    B, N, _ = xyz.shape
    x = xyz[:, :, 0]
    y = xyz[:, :, 1]
    z = xyz[:, :, 2]
    col = lambda a: a[:, :, None]
    row = lambda a: a[:, None, :]
    r2 = jnp.reshape(radius, (1, 1))

    col_spec = pl.BlockSpec((1, BI, 1), lambda b, i: (b, i, 0))
    row_spec = pl.BlockSpec((1, 1, N), lambda b, i: (b, 0, 0))
    grid = (B, N // BI)
    idx, gidx, gxyz24 = pl.pallas_call(
        _select_body,
        grid=grid,
        in_specs=[
            pl.BlockSpec(memory_space=pltpu.SMEM),
            col_spec, col_spec, col_spec,
            row_spec, row_spec, row_spec,
        ],
        out_specs=[
            pl.BlockSpec((1, BI, 8), lambda b, i: (b, i, 0)),
            pl.BlockSpec((1, BI, 8), lambda b, i: (b, i, 0)),
            pl.BlockSpec((1, BI, 24), lambda b, i: (b, i, 0)),
        ],
        out_shape=[
            jax.ShapeDtypeStruct((B, N, 8), jnp.int32),
            jax.ShapeDtypeStruct((B, N, 8), jnp.int32),
            jax.ShapeDtypeStruct((B, N, 24), jnp.float32),
        ],
    )(r2, col(x), col(y), col(z), row(x), row(y), row(z))
    return idx, gidx, gxyz24


def _sc_gather(table, idxs):
    R, D = table.shape
    M = idxs.shape[0]
    info = plsc.get_sparse_core_info()
    nw = info.num_cores * info.num_subcores
    per_w = M // nw
    CH = 128
    T = per_w // CH
    idx3 = idxs.reshape(nw, T, CH)
    mesh = plsc.VectorSubcoreMesh(core_axis_name="c", subcore_axis_name="s")

    @functools.partial(
        pl.kernel,
        mesh=mesh,
        compiler_params=pltpu.CompilerParams(use_tc_tiling_on_sc=False),
        out_type=jax.ShapeDtypeStruct((M, D), jnp.float32),
        scratch_types=[
            pltpu.VMEM((T, CH), jnp.int32),
            pltpu.VMEM((CH, D), jnp.float32),
            pltpu.VMEM((CH, D), jnp.float32),
            pltpu.SemaphoreType.DMA,
            pltpu.SemaphoreType.DMA,
        ],
    )
    def k(idx_hbm, table_hbm, out_hbm, idx_v, rows0, rows1, sem0, sem1):
        wid = lax.axis_index("s") * info.num_cores + lax.axis_index("c")
        pltpu.sync_copy(idx_hbm.at[wid], idx_v)
        bufs = (rows0, rows1)
        sems = (sem0, sem1)
        cps = [None, None]
        cps[0] = pltpu.async_copy(table_hbm.at[idx_v.at[0]], bufs[0], sems[0])
        for t in range(T):
            cur = t % 2
            nxt = (t + 1) % 2
            if t + 1 < T:
                cps[nxt] = pltpu.async_copy(
                    table_hbm.at[idx_v.at[t + 1]], bufs[nxt], sems[nxt])
            cps[cur].wait()
            pltpu.sync_copy(bufs[cur], out_hbm.at[pl.ds(wid * per_w + t * CH, CH)])

    return k(idx3, table)


def kernel(xyz, points, radius):
    B, N, _ = xyz.shape
    P = points.shape[-1]
    idx, gidx, gxyz24 = _select_cube_tc(xyz, radius)
    grouped_xyz = gxyz24.reshape(B, N, 8, 3)
    gp = _sc_gather(points.reshape(B * N, P), gidx.reshape(B * N * 8))
    grouped_points = jnp.concatenate(
        [grouped_xyz, gp.reshape(B, N, 8, P)], axis=-1)
    return grouped_xyz, grouped_points, idx


# speedup vs baseline: 1.9643x; 1.2470x over previous
import functools

import jax
import jax.numpy as jnp
from jax import lax
from jax.experimental import pallas as pl
from jax.experimental.pallas import tpu as pltpu
from jax.experimental.pallas import tpu_sc as plsc

BI = 256


def _select_body(r_ref, colx, coly, colz, rowx, rowy, rowz, xyzt_ref,
                 idx_ref, gidx_ref, gxyz_ref):
    b = pl.program_id(0)
    ib = pl.program_id(1)
    r = r_ref[0, 0]
    n = colx.shape[-2]
    bi = rowx.shape[-1]

    xj = colx[0]
    yj = coly[0]
    zj = colz[0]
    xi = rowx[0]
    yi = rowy[0]
    zi = rowz[0]

    dx = xj - xi
    dy = yj - yi
    dz = zj - zi
    dist2 = dx * dx + dy * dy + dz * dz

    jota = lax.broadcasted_iota(jnp.int32, (n, bi), 0)
    iglob = lax.broadcasted_iota(jnp.int32, (n, bi), 1) + ib * bi
    within = ((jnp.abs(dx) < r) & (jnp.abs(dy) < r) & (jnp.abs(dz) < r)
              & (jota != iglob))
    inf = jnp.float32(1e10)
    d_in = jnp.where(within, dist2, inf)
    iglob_row = lax.broadcasted_iota(jnp.int32, (1, bi), 1) + ib * bi
    octant = ((dx > 0).astype(jnp.int32) * 4
              + (dy > 0).astype(jnp.int32) * 2
              + (dz > 0).astype(jnp.int32))

    xyzt = xyzt_ref[0]
    h1 = xyzt.astype(jnp.bfloat16)
    r1 = xyzt - h1.astype(jnp.float32)
    h2 = r1.astype(jnp.bfloat16)
    h3 = (r1 - h2.astype(jnp.float32)).astype(jnp.bfloat16)
    center = jnp.concatenate([xi, yi, zi], axis=0)
    dn = (((1,), (0,)), ((), ()))

    for o in range(8):
        d_o = jnp.where(octant == o, d_in, inf)
        mn = jnp.min(d_o, axis=0, keepdims=True)
        cand = jnp.min(jnp.where(d_o == mn, jota, n), axis=0, keepdims=True)
        sel = jnp.where(mn < 1e9, cand, iglob_row)
        idx_ref[0, o:o + 1, :] = sel
        gidx_ref[0, o:o + 1, :] = sel + b * n
        oh = (jota == sel).astype(jnp.bfloat16)
        g = (jax.lax.dot_general(h1, oh, dn, preferred_element_type=jnp.float32)
             + jax.lax.dot_general(h2, oh, dn, preferred_element_type=jnp.float32)
             + jax.lax.dot_general(h3, oh, dn, preferred_element_type=jnp.float32))
        gxyz_ref[0, 3 * o:3 * o + 3, :] = g - center


def _select_cube_tc(xyz, radius):
    B, N, _ = xyz.shape
    x = xyz[:, :, 0]
    y = xyz[:, :, 1]
    z = xyz[:, :, 2]
    col = lambda a: a[:, :, None]
    row = lambda a: a[:, None, :]
    xyzt = jnp.stack([x, y, z], axis=1)
    r2 = jnp.reshape(radius, (1, 1))

    col_spec = pl.BlockSpec((1, N, 1), lambda b, i: (b, 0, 0))
    row_spec = pl.BlockSpec((1, 1, BI), lambda b, i: (b, 0, i))
    grid = (B, N // BI)
    idx_t, gidx_t, gxyz_t = pl.pallas_call(
        _select_body,
        grid=grid,
        in_specs=[
            pl.BlockSpec(memory_space=pltpu.SMEM),
            col_spec, col_spec, col_spec,
            row_spec, row_spec, row_spec,
            pl.BlockSpec((1, 3, N), lambda b, i: (b, 0, 0)),
        ],
        out_specs=[
            pl.BlockSpec((1, 8, BI), lambda b, i: (b, 0, i)),
            pl.BlockSpec((1, 8, BI), lambda b, i: (b, 0, i)),
            pl.BlockSpec((1, 24, BI), lambda b, i: (b, 0, i)),
        ],
        out_shape=[
            jax.ShapeDtypeStruct((B, 8, N), jnp.int32),
            jax.ShapeDtypeStruct((B, 8, N), jnp.int32),
            jax.ShapeDtypeStruct((B, 24, N), jnp.float32),
        ],
    )(r2, col(x), col(y), col(z), row(x), row(y), row(z), xyzt)
    return idx_t, gidx_t, gxyz_t


def _sc_gather(table, idxs):
    R, D = table.shape
    M = idxs.shape[0]
    info = plsc.get_sparse_core_info()
    nw = info.num_cores * info.num_subcores
    per_w = M // nw
    CH = 128
    T = per_w // CH
    idx3 = idxs.reshape(nw, T, CH)
    mesh = plsc.VectorSubcoreMesh(core_axis_name="c", subcore_axis_name="s")

    @functools.partial(
        pl.kernel,
        mesh=mesh,
        compiler_params=pltpu.CompilerParams(use_tc_tiling_on_sc=False),
        out_type=jax.ShapeDtypeStruct((M, D), jnp.float32),
        scratch_types=[
            pltpu.VMEM((T, CH), jnp.int32),
            pltpu.VMEM((CH, D), jnp.float32),
            pltpu.VMEM((CH, D), jnp.float32),
            pltpu.SemaphoreType.DMA,
            pltpu.SemaphoreType.DMA,
        ],
    )
    def k(idx_hbm, table_hbm, out_hbm, idx_v, rows0, rows1, sem0, sem1):
        wid = lax.axis_index("s") * info.num_cores + lax.axis_index("c")
        pltpu.sync_copy(idx_hbm.at[wid], idx_v)
        bufs = (rows0, rows1)
        sems = (sem0, sem1)
        cps = [None, None]
        cps[0] = pltpu.async_copy(table_hbm.at[idx_v.at[0]], bufs[0], sems[0])
        for t in range(T):
            cur = t % 2
            nxt = (t + 1) % 2
            if t + 1 < T:
                cps[nxt] = pltpu.async_copy(
                    table_hbm.at[idx_v.at[t + 1]], bufs[nxt], sems[nxt])
            cps[cur].wait()
            pltpu.sync_copy(bufs[cur], out_hbm.at[pl.ds(wid * per_w + t * CH, CH)])

    return k(idx3, table)


def kernel(xyz, points, radius):
    B, N, _ = xyz.shape
    P = points.shape[-1]
    idx_t, gidx_t, gxyz_t = _select_cube_tc(xyz, radius)
    idx = jnp.transpose(idx_t, (0, 2, 1))
    gidx = jnp.transpose(gidx_t, (0, 2, 1)).reshape(B * N * 8)
    grouped_xyz = jnp.transpose(gxyz_t, (0, 2, 1)).reshape(B, N, 8, 3)
    gp = _sc_gather(points.reshape(B * N, P), gidx)
    grouped_points = jnp.concatenate(
        [grouped_xyz, gp.reshape(B, N, 8, P)], axis=-1)
    return grouped_xyz, grouped_points, idx


# speedup vs baseline: 2.0565x; 1.0470x over previous
import functools

import jax
import jax.numpy as jnp
from jax import lax
from jax.experimental import pallas as pl
from jax.experimental.pallas import tpu as pltpu
from jax.experimental.pallas import tpu_sc as plsc

BI = 512


def _select_body(r_ref, colx, coly, colz, rowx, rowy, rowz, xyzt_ref,
                 idx_ref, gidx_ref, gxyz_ref):
    b = pl.program_id(0)
    ib = pl.program_id(1)
    r = r_ref[0, 0]
    n = colx.shape[-2]
    bi = rowx.shape[-1]

    xj = colx[0]
    yj = coly[0]
    zj = colz[0]
    xi = rowx[0]
    yi = rowy[0]
    zi = rowz[0]

    dx = xj - xi
    dy = yj - yi
    dz = zj - zi
    dist2 = dx * dx + dy * dy + dz * dz

    jota = lax.broadcasted_iota(jnp.int32, (n, bi), 0)
    jota_f = jota.astype(jnp.float32)
    iglob = lax.broadcasted_iota(jnp.int32, (n, bi), 1) + ib * bi
    within = ((jnp.abs(dx) < r) & (jnp.abs(dy) < r) & (jnp.abs(dz) < r)
              & (jota != iglob))
    inf = jnp.float32(1e10)
    d_in = jnp.where(within, dist2, inf)
    iglob_row = lax.broadcasted_iota(jnp.int32, (1, bi), 1) + ib * bi
    octant = ((dx > 0).astype(jnp.int32) * 4
              + (dy > 0).astype(jnp.int32) * 2
              + (dz > 0).astype(jnp.int32))

    xyzt = xyzt_ref[0]
    h1 = xyzt.astype(jnp.bfloat16)
    r1 = xyzt - h1.astype(jnp.float32)
    h2 = r1.astype(jnp.bfloat16)
    h3 = (r1 - h2.astype(jnp.float32)).astype(jnp.bfloat16)
    center = jnp.concatenate([xi, yi, zi], axis=0)
    dn = (((1,), (0,)), ((), ()))

    for o in range(8):
        d_o = jnp.where(octant == o, d_in, inf)
        mn = jnp.min(d_o, axis=0, keepdims=True)
        cand_f = jnp.min(jnp.where(d_o == mn, jota_f, jnp.float32(n)),
                         axis=0, keepdims=True)
        cand = cand_f.astype(jnp.int32)
        sel = jnp.where(mn < 1e9, cand, iglob_row)
        idx_ref[0, o:o + 1, :] = sel
        gidx_ref[0, o:o + 1, :] = sel + b * n
        oh = (jota == sel).astype(jnp.bfloat16)
        g = (jax.lax.dot_general(h1, oh, dn, preferred_element_type=jnp.float32)
             + jax.lax.dot_general(h2, oh, dn, preferred_element_type=jnp.float32)
             + jax.lax.dot_general(h3, oh, dn, preferred_element_type=jnp.float32))
        gxyz_ref[0, 3 * o:3 * o + 3, :] = g - center


def _select_cube_tc(xyz, radius):
    B, N, _ = xyz.shape
    x = xyz[:, :, 0]
    y = xyz[:, :, 1]
    z = xyz[:, :, 2]
    col = lambda a: a[:, :, None]
    row = lambda a: a[:, None, :]
    xyzt = jnp.stack([x, y, z], axis=1)
    r2 = jnp.reshape(radius, (1, 1))

    col_spec = pl.BlockSpec((1, N, 1), lambda b, i: (b, 0, 0))
    row_spec = pl.BlockSpec((1, 1, BI), lambda b, i: (b, 0, i))
    grid = (B, N // BI)
    idx_t, gidx_t, gxyz_t = pl.pallas_call(
        _select_body,
        grid=grid,
        in_specs=[
            pl.BlockSpec(memory_space=pltpu.SMEM),
            col_spec, col_spec, col_spec,
            row_spec, row_spec, row_spec,
            pl.BlockSpec((1, 3, N), lambda b, i: (b, 0, 0)),
        ],
        out_specs=[
            pl.BlockSpec((1, 8, BI), lambda b, i: (b, 0, i)),
            pl.BlockSpec((1, 8, BI), lambda b, i: (b, 0, i)),
            pl.BlockSpec((1, 24, BI), lambda b, i: (b, 0, i)),
        ],
        out_shape=[
            jax.ShapeDtypeStruct((B, 8, N), jnp.int32),
            jax.ShapeDtypeStruct((B, 8, N), jnp.int32),
            jax.ShapeDtypeStruct((B, 24, N), jnp.float32),
        ],
    )(r2, col(x), col(y), col(z), row(x), row(y), row(z), xyzt)
    return idx_t, gidx_t, gxyz_t


def _sc_gather(table, idxs):
    R, D = table.shape
    M = idxs.shape[0]
    info = plsc.get_sparse_core_info()
    nw = info.num_cores * info.num_subcores
    per_w = M // nw
    CH = 128
    T = per_w // CH
    idx3 = idxs.reshape(nw, T, CH)
    mesh = plsc.VectorSubcoreMesh(core_axis_name="c", subcore_axis_name="s")

    @functools.partial(
        pl.kernel,
        mesh=mesh,
        compiler_params=pltpu.CompilerParams(use_tc_tiling_on_sc=False),
        out_type=jax.ShapeDtypeStruct((M, D), jnp.float32),
        scratch_types=[
            pltpu.VMEM((T, CH), jnp.int32),
            pltpu.VMEM((CH, D), jnp.float32),
            pltpu.VMEM((CH, D), jnp.float32),
            pltpu.SemaphoreType.DMA,
            pltpu.SemaphoreType.DMA,
        ],
    )
    def k(idx_hbm, table_hbm, out_hbm, idx_v, rows0, rows1, sem0, sem1):
        wid = lax.axis_index("s") * info.num_cores + lax.axis_index("c")
        pltpu.sync_copy(idx_hbm.at[wid], idx_v)
        bufs = (rows0, rows1)
        sems = (sem0, sem1)
        cps = [None, None]
        cps[0] = pltpu.async_copy(table_hbm.at[idx_v.at[0]], bufs[0], sems[0])
        for t in range(T):
            cur = t % 2
            nxt = (t + 1) % 2
            if t + 1 < T:
                cps[nxt] = pltpu.async_copy(
                    table_hbm.at[idx_v.at[t + 1]], bufs[nxt], sems[nxt])
            cps[cur].wait()
            pltpu.sync_copy(bufs[cur], out_hbm.at[pl.ds(wid * per_w + t * CH, CH)])

    return k(idx3, table)


def kernel(xyz, points, radius):
    B, N, _ = xyz.shape
    P = points.shape[-1]
    idx_t, gidx_t, gxyz_t = _select_cube_tc(xyz, radius)
    idx = jnp.transpose(idx_t, (0, 2, 1))
    gidx = jnp.transpose(gidx_t, (0, 2, 1)).reshape(B * N * 8)
    grouped_xyz = jnp.transpose(gxyz_t, (0, 2, 1)).reshape(B, N, 8, 3)
    gp = _sc_gather(points.reshape(B * N, P), gidx)
    grouped_points = jnp.concatenate(
        [grouped_xyz, gp.reshape(B, N, 8, P)], axis=-1)
    return grouped_xyz, grouped_points, idx
